# baseline (speedup 1.0000x reference)
import jax
import jax.numpy as jnp
from jax.experimental import pallas as pl
from jax.experimental.pallas import tpu as pltpu
from jax.experimental.pallas import tpu_sc as plsc


_LANES = 16
_BM = 8


def _tc_add_kernel(x_ref, pe_ref, o_ref):
    o_ref[...] = x_ref[...] + pe_ref[...][None]


def _tc_kernel(x, pe_weight):
    B, S, D = x.shape
    BS = 512
    return pl.pallas_call(
        _tc_add_kernel,
        grid=(S // BS,),
        in_specs=[
            pl.BlockSpec((B, BS, D), lambda s: (0, s, 0)),
            pl.BlockSpec((BS, D), lambda s: (s, 0)),
        ],
        out_specs=pl.BlockSpec((B, BS, D), lambda s: (0, s, 0)),
        out_shape=jax.ShapeDtypeStruct((B, S, D), x.dtype),
    )(x, pe_weight[:S])


_BLK = 16384


def _sc_kernel(x1d, pe1d, batch):
    N = x1d.shape[0]
    P = pe1d.shape[0]
    n_seq_blocks = P // _BLK

    mesh = plsc.VectorSubcoreMesh(core_axis_name="core",
                                  subcore_axis_name="subcore")

    @pl.kernel(out_type=jax.ShapeDtypeStruct((N,), x1d.dtype), mesh=mesh,
               scratch_types=[])
    def sc_add(x_hbm, pe_hbm, o_hbm):
        def body(x_vmem, pe_vmem, o_vmem):
            @plsc.parallel_loop(0, _BLK, step=_LANES, unroll=8)
            def _(k):
                sl = pl.ds(k, _LANES)
                o_vmem.at[sl][...] = x_vmem.at[sl][...] + pe_vmem.at[sl][...]

        pltpu.emit_pipeline(
            body,
            grid=(n_seq_blocks, batch),
            in_specs=[
                pl.BlockSpec((_BLK,), index_map=lambda i, b: (b * n_seq_blocks + i,)),
                pl.BlockSpec((_BLK,), index_map=lambda i, b: (i,)),
            ],
            out_specs=[pl.BlockSpec((_BLK,), index_map=lambda i, b: (b * n_seq_blocks + i,))],
            core_axis_name=("core", "subcore"),
            dimension_semantics=(pltpu.PARALLEL, pltpu.ARBITRARY),
        )(x_hbm, pe_hbm, o_hbm)

    return sc_add(x1d, pe1d)


def kernel(x, pe_weight):
    B, S, D = x.shape
    out1d = _sc_kernel(x.reshape(B * S * D), pe_weight[:S].reshape(S * D), B)
    return out1d.reshape(B, S, D)


# speedup vs baseline: 2.9458x; 2.9458x over previous
import jax
import jax.numpy as jnp
from jax.experimental import pallas as pl
from jax.experimental.pallas import tpu as pltpu
from jax.experimental.pallas import tpu_sc as plsc


_LANES = 16
_BM = 8


def _tc_add_kernel(x_ref, pe_ref, o_ref):
    o_ref[...] = x_ref[...] + pe_ref[...][None]


def _tc_kernel(x, pe_weight):
    B, S, D = x.shape
    BS = 512
    return pl.pallas_call(
        _tc_add_kernel,
        grid=(S // BS,),
        in_specs=[
            pl.BlockSpec((B, BS, D), lambda s: (0, s, 0)),
            pl.BlockSpec((BS, D), lambda s: (s, 0)),
        ],
        out_specs=pl.BlockSpec((B, BS, D), lambda s: (0, s, 0)),
        out_shape=jax.ShapeDtypeStruct((B, S, D), x.dtype),
    )(x, pe_weight[:S])


_BMR = 16


def _sc_kernel(x2d, pe, batch):
    R, D = x2d.shape
    S = pe.shape[0]
    n_seq_blocks = S // _BMR

    mesh = plsc.VectorSubcoreMesh(core_axis_name="core",
                                  subcore_axis_name="subcore")

    @pl.kernel(out_type=jax.ShapeDtypeStruct((R, D), x2d.dtype), mesh=mesh,
               scratch_types=[])
    def sc_add(x_hbm, pe_hbm, o_hbm):
        def body(x_vmem, pe_vmem, o_vmem):
            @pl.loop(0, _BMR)
            def _(r):
                @plsc.parallel_loop(0, D, step=_LANES, unroll=8)
                def _(c):
                    sl = (pl.ds(r, 1), pl.ds(c, _LANES))
                    o_vmem.at[*sl][...] = x_vmem.at[*sl][...] + pe_vmem.at[*sl][...]

        pltpu.emit_pipeline(
            body,
            grid=(n_seq_blocks, batch),
            in_specs=[
                pl.BlockSpec((_BMR, D), index_map=lambda i, b: (b * n_seq_blocks + i, 0)),
                pl.BlockSpec((_BMR, D), index_map=lambda i, b: (i, 0)),
            ],
            out_specs=[pl.BlockSpec((_BMR, D), index_map=lambda i, b: (b * n_seq_blocks + i, 0))],
            core_axis_name=("core", "subcore"),
            dimension_semantics=(pltpu.PARALLEL, pltpu.ARBITRARY),
        )(x_hbm, pe_hbm, o_hbm)

    return sc_add(x2d, pe)


def kernel(x, pe_weight):
    B, S, D = x.shape
    out2d = _sc_kernel(x.reshape(B * S, D), pe_weight[:S], B)
    return out2d.reshape(B, S, D)


# speedup vs baseline: 3.1259x; 1.0611x over previous
import jax
import jax.numpy as jnp
from jax.experimental import pallas as pl
from jax.experimental.pallas import tpu as pltpu
from jax.experimental.pallas import tpu_sc as plsc


_LANES = 16
_BM = 8


def _tc_add_kernel(x_ref, pe_ref, o_ref):
    o_ref[...] = x_ref[...] + pe_ref[...][None]


def _tc_kernel(x, pe_weight):
    B, S, D = x.shape
    BS = 512
    return pl.pallas_call(
        _tc_add_kernel,
        grid=(S // BS,),
        in_specs=[
            pl.BlockSpec((B, BS, D), lambda s: (0, s, 0)),
            pl.BlockSpec((BS, D), lambda s: (s, 0)),
        ],
        out_specs=pl.BlockSpec((B, BS, D), lambda s: (0, s, 0)),
        out_shape=jax.ShapeDtypeStruct((B, S, D), x.dtype),
    )(x, pe_weight[:S])


_BMR = 4


def _sc_kernel(x, pe):
    B, S, D = x.shape
    n_seq_blocks = S // _BMR

    mesh = plsc.VectorSubcoreMesh(core_axis_name="core",
                                  subcore_axis_name="subcore")

    @pl.kernel(out_type=jax.ShapeDtypeStruct((B, S, D), x.dtype), mesh=mesh,
               scratch_types=[])
    def sc_add(x_hbm, pe_hbm, o_hbm):
        def body(x_vmem, pe_vmem, o_vmem):
            @pl.loop(0, _BMR)
            def _(r):
                @plsc.parallel_loop(0, D, step=_LANES, unroll=8)
                def _(c):
                    sl = (pl.ds(r, 1), pl.ds(c, _LANES))
                    p = pe_vmem.at[*sl][...]
                    for b in range(B):
                        o_vmem.at[b, *sl][...] = x_vmem.at[b, *sl][...] + p

        pltpu.emit_pipeline(
            body,
            grid=(n_seq_blocks,),
            in_specs=[
                pl.BlockSpec((B, _BMR, D), index_map=lambda i: (0, i, 0)),
                pl.BlockSpec((_BMR, D), index_map=lambda i: (i, 0)),
            ],
            out_specs=[pl.BlockSpec((B, _BMR, D), index_map=lambda i: (0, i, 0))],
            core_axis_name=("core", "subcore"),
            dimension_semantics=(pltpu.PARALLEL,),
        )(x_hbm, pe_hbm, o_hbm)

    return sc_add(x, pe)


def kernel(x, pe_weight):
    B, S, D = x.shape
    return _sc_kernel(x, pe_weight[:S])
